# Initial kernel scaffold; baseline (speedup 1.0000x reference)
import jax
import jax.numpy as jnp
from jax.experimental import pallas as pl


def kernel(g1_x, g1_ent, g2_x, g2_ent, params):
    raise NotImplementedError("write your pallas kernel here")


# speedup vs baseline: 2.4438x; 2.4438x over previous
import functools

import jax
import jax.numpy as jnp
from jax.experimental import pallas as pl

N_EMBD = 384
GRID = 10
BLK = 400


def _elu(x):
    return jnp.where(x > 0, x, jnp.exp(jnp.minimum(x, 0.0)) - 1.0)


def _erf(x):
    a1, a2, a3, a4, a5 = (0.254829592, -0.284496736, 1.421413741,
                          -1.453152027, 1.061405429)
    p = 0.3275911
    s = jnp.sign(x)
    z = jnp.abs(x)
    t = 1.0 / (1.0 + p * z)
    poly = t * (a1 + t * (a2 + t * (a3 + t * (a4 + t * a5))))
    y = 1.0 - poly * jnp.exp(-z * z)
    return s * y


def _gelu(x):
    return 0.5 * x * (1.0 + _erf(x * 0.7071067811865476))


def _shift_down(x3):
    z = jnp.zeros_like(x3[:1])
    return jnp.concatenate([z, x3[:-1]], axis=0)


def _shift_up(x3):
    z = jnp.zeros_like(x3[:1])
    return jnp.concatenate([x3[1:], z], axis=0)


def _body(g3_ref, ent_ref, w1a, w1b, w1c, b1, w2a, w2b, w2c, b2,
          wl, bl, tb, tp, lnw, lnb, wf, bf, wp, bp, out_ref):
    B = g3_ref.shape[1]
    g3 = g3_ref[...]
    gflat = g3.reshape(GRID * B, 6)
    h = (jnp.dot(_shift_down(g3).reshape(GRID * B, 6), w1a[...],
                 preferred_element_type=jnp.float32)
         + jnp.dot(gflat, w1b[...], preferred_element_type=jnp.float32)
         + jnp.dot(_shift_up(g3).reshape(GRID * B, 6), w1c[...],
                   preferred_element_type=jnp.float32)
         + b1[...])
    h = _elu(h)
    h3 = h.reshape(GRID, B, N_EMBD)
    y = (jnp.dot(_shift_down(h3).reshape(GRID * B, N_EMBD), w2a[...],
                 preferred_element_type=jnp.float32)
         + jnp.dot(h, w2b[...], preferred_element_type=jnp.float32)
         + jnp.dot(_shift_up(h3).reshape(GRID * B, N_EMBD), w2c[...],
                   preferred_element_type=jnp.float32)
         + b2[...])
    p3 = _elu(y).reshape(GRID, B, N_EMBD)
    pooled = jnp.sum(p3, axis=0) * (1.0 / GRID)
    g = jnp.dot(pooled, wl[...], preferred_element_type=jnp.float32) + bl[...]

    ent = ent_ref[...]
    it = jax.lax.broadcasted_iota(jnp.int32, (B, tb.shape[0]), 1)
    oh = ((it == ent[:, 1:2]).astype(jnp.float32)
          + (it == ent[:, 2:3] + 6).astype(jnp.float32)
          + (it == ent[:, 3:4] + 262).astype(jnp.float32))
    emb = jnp.dot(oh, tb[...], preferred_element_type=jnp.float32)
    itp = jax.lax.broadcasted_iota(jnp.int32, (B, tp.shape[0]), 1)
    pts = []
    for j in range(3):
        ohp = ((itp == ent[:, 4 + j:5 + j]).astype(jnp.float32)
               + (itp == ent[:, 7 + j:8 + j] + 256).astype(jnp.float32)
               + (itp == ent[:, 10 + j:11 + j] + 512).astype(jnp.float32))
        pts.append(jnp.dot(ohp, tp[...], preferred_element_type=jnp.float32))
    epts = jnp.concatenate(pts, axis=1)

    mask = (ent[:, 0:1] <= 0).astype(jnp.float32)
    x = mask * (g + emb + epts)

    mu = jnp.mean(x, axis=1, keepdims=True)
    var = jnp.mean((x - mu) ** 2, axis=1, keepdims=True)
    xn = (x - mu) * jax.lax.rsqrt(var + 1e-5) * lnw[...] + lnb[...]
    hh = jnp.dot(xn, wf[...], preferred_element_type=jnp.float32) + bf[...]
    hh = _gelu(hh)
    out_ref[...] = x + jnp.dot(hh, wp[...],
                               preferred_element_type=jnp.float32) + bp[...]


def kernel(g1_x, g1_ent, g2_x, g2_ent, params):
    p = params
    n1 = g1_x.shape[0]
    n2 = g2_x.shape[0]
    n = n1 + n2
    xg = jnp.concatenate([g1_x[:, 0], g2_x[:, 0]], axis=0)
    g3 = jnp.transpose(xg, (1, 0, 2))
    ent = jnp.concatenate([g1_ent, g2_ent], axis=0).astype(jnp.int32)

    w1 = p['conv1_w']
    w1a = jnp.transpose(w1[:, :, 0])
    w1b = jnp.transpose(w1[:, :, 1])
    w1c = jnp.transpose(w1[:, :, 2])
    w2 = p['conv2_w']
    w2a = jnp.transpose(w2[:, :, 0])
    w2b = jnp.transpose(w2[:, :, 1])
    w2c = jnp.transpose(w2[:, :, 2])
    wl = jnp.transpose(p['grid_lin_w'])
    tb = jnp.concatenate([p['emb_entity_types'], p['emb_length'],
                          p['emb_radius'],
                          jnp.zeros((2, N_EMBD), jnp.float32)], axis=0)
    tp = jnp.concatenate([p['emb_start_point'], p['emb_middle_point'],
                          p['emb_end_point']], axis=0)
    wf = jnp.transpose(p['fc_w'])
    wp = jnp.transpose(p['proj_w'])

    row = lambda a: a.reshape(1, -1)
    nblk = n // BLK
    const = lambda ndim: pl.BlockSpec(index_map=lambda i: (0,) * ndim)
    out = pl.pallas_call(
        _body,
        grid=(nblk,),
        in_specs=[
            pl.BlockSpec((GRID, BLK, 6), lambda i: (0, i, 0)),
            pl.BlockSpec((BLK, 13), lambda i: (i, 0)),
            const(2), const(2), const(2), const(2),
            const(2), const(2), const(2), const(2),
            const(2), const(2), const(2), const(2),
            const(2), const(2), const(2), const(2), const(2), const(2),
        ],
        out_specs=pl.BlockSpec((BLK, N_EMBD), lambda i: (i, 0)),
        out_shape=jax.ShapeDtypeStruct((n, N_EMBD), jnp.float32),
    )(g3, ent, w1a, w1b, w1c, row(p['conv1_b']), w2a, w2b, w2c,
      row(p['conv2_b']), wl, row(p['grid_lin_b']), tb, tp,
      row(p['ln_w']), row(p['ln_b']), wf, row(p['fc_b']), wp,
      row(p['proj_b']))
    return (out[:n1], out[n1:])
